# baseline (speedup 1.0000x reference)
import functools

import jax
import jax.numpy as jnp
from jax import lax
from jax.experimental import pallas as pl
from jax.experimental.pallas import tpu as pltpu
from jax.experimental.pallas import tpu_sc as plsc

_N_PAPER, _N_AUTHOR, _N_FIELD = 100000, 50000, 10000
_EMBED = 128
_ROWS_W = 5000
_CHUNK = 200
_NCHUNK = _ROWS_W // _CHUNK
_TABLES = ((0, 20), (20, 10), (30, 2))


def _worker_copy(src, dst, wid, wstart, bufs, sems):
    base = (wid - wstart) * _ROWS_W

    def in_copy(k):
        return pltpu.make_async_copy(
            src.at[pl.ds(base + k * _CHUNK, _CHUNK), :],
            bufs[k % 2],
            sems[k % 2],
        )

    def out_copy(k):
        return pltpu.make_async_copy(
            bufs[k % 2],
            dst.at[pl.ds(base + k * _CHUNK, _CHUNK), :],
            sems[2 + (k % 2)],
        )

    in_copy(0).start()
    for k in range(_NCHUNK):
        if k + 1 < _NCHUNK:
            if k >= 1:
                out_copy(k - 1).wait()
            in_copy(k + 1).start()
        in_copy(k).wait()
        out_copy(k).start()
    if _NCHUNK >= 2:
        out_copy(_NCHUNK - 2).wait()
    out_copy(_NCHUNK - 1).wait()


def _sc_copy(p_in, a_in, f_in, p_out, a_out, f_out,
             buf0, buf1, s0, s1, s2, s3):
    nc = 2
    wid = lax.axis_index("s") * nc + lax.axis_index("c")
    bufs = (buf0, buf1)
    sems = (s0, s1, s2, s3)
    srcs = (p_in, a_in, f_in)
    dsts = (p_out, a_out, f_out)
    for t, (wstart, nworkers) in enumerate(_TABLES):
        @pl.when(jnp.logical_and(wid >= wstart, wid < wstart + nworkers))
        def _(t=t, wstart=wstart):
            _worker_copy(srcs[t], dsts[t], wid, wstart, bufs, sems)


def kernel(embed_paper, embed_author, embed_field):
    mesh = plsc.VectorSubcoreMesh(core_axis_name="c", subcore_axis_name="s")
    run = pl.kernel(
        _sc_copy,
        out_type=tuple(
            jax.ShapeDtypeStruct(x.shape, x.dtype)
            for x in (embed_paper, embed_author, embed_field)
        ),
        mesh=mesh,
        scratch_types=[
            pltpu.VMEM((_CHUNK, _EMBED), jnp.float32),
            pltpu.VMEM((_CHUNK, _EMBED), jnp.float32),
            pltpu.SemaphoreType.DMA,
            pltpu.SemaphoreType.DMA,
            pltpu.SemaphoreType.DMA,
            pltpu.SemaphoreType.DMA,
        ],
    )
    return run(embed_paper, embed_author, embed_field)


# speedup vs baseline: 1.1446x; 1.1446x over previous
import jax
import jax.numpy as jnp
from jax import lax
from jax.experimental import pallas as pl
from jax.experimental.pallas import tpu as pltpu
from jax.experimental.pallas import tpu_sc as plsc

_EMBED = 128


_TC_CHUNK = 12500
_TC_SLOTS = 4


def _tc_pipeline(src, dst, *scratch):
    bufs = scratch[:_TC_SLOTS]
    sins = scratch[_TC_SLOTS:2 * _TC_SLOTS]
    souts = scratch[2 * _TC_SLOTS:]
    n = src.shape[0] // _TC_CHUNK
    depth = _TC_SLOTS // 2

    def in_copy(i):
        return pltpu.make_async_copy(
            src.at[pl.ds(i * _TC_CHUNK, _TC_CHUNK), :],
            bufs[i % _TC_SLOTS],
            sins[i % _TC_SLOTS],
        )

    def out_copy(i):
        return pltpu.make_async_copy(
            bufs[i % _TC_SLOTS],
            dst.at[pl.ds(i * _TC_CHUNK, _TC_CHUNK), :],
            souts[i % _TC_SLOTS],
        )

    for i in range(min(depth, n)):
        in_copy(i).start()
    for i in range(n):
        j = i + depth
        if j < n:
            if j - _TC_SLOTS >= 0:
                out_copy(j - _TC_SLOTS).wait()
            in_copy(j).start()
        in_copy(i).wait()
        out_copy(i).start()
    for i in range(max(0, n - 2 * depth), n):
        out_copy(i).wait()


def _tc_copy(x):
    return pl.pallas_call(
        _tc_pipeline,
        in_specs=[pl.BlockSpec(memory_space=pltpu.MemorySpace.HBM)],
        out_specs=pl.BlockSpec(memory_space=pltpu.MemorySpace.HBM),
        out_shape=jax.ShapeDtypeStruct(x.shape, x.dtype),
        scratch_shapes=(
            [pltpu.VMEM((_TC_CHUNK, _EMBED), jnp.float32)] * _TC_SLOTS
            + [pltpu.SemaphoreType.DMA] * (2 * _TC_SLOTS)
        ),
    )(x)


_SC_ROWS_W = 2000
_SC_CHUNK = 200
_SC_NCHUNK = _SC_ROWS_W // _SC_CHUNK
_SC_TABLES = ((0, 25), (25, 5))


def _sc_worker_copy(src, dst, wid, wstart, bufs, sems):
    base = (wid - wstart) * _SC_ROWS_W

    def in_copy(k):
        return pltpu.make_async_copy(
            src.at[pl.ds(base + k * _SC_CHUNK, _SC_CHUNK), :],
            bufs[k % 2],
            sems[k % 2],
        )

    def out_copy(k):
        return pltpu.make_async_copy(
            bufs[k % 2],
            dst.at[pl.ds(base + k * _SC_CHUNK, _SC_CHUNK), :],
            sems[2 + (k % 2)],
        )

    in_copy(0).start()
    for k in range(_SC_NCHUNK):
        if k + 1 < _SC_NCHUNK:
            if k >= 1:
                out_copy(k - 1).wait()
            in_copy(k + 1).start()
        in_copy(k).wait()
        out_copy(k).start()
    if _SC_NCHUNK >= 2:
        out_copy(_SC_NCHUNK - 2).wait()
    out_copy(_SC_NCHUNK - 1).wait()


def _sc_body(a_in, f_in, a_out, f_out, buf0, buf1, s0, s1, s2, s3):
    wid = lax.axis_index("s") * 2 + lax.axis_index("c")
    bufs = (buf0, buf1)
    sems = (s0, s1, s2, s3)
    srcs = (a_in, f_in)
    dsts = (a_out, f_out)
    for t, (wstart, nworkers) in enumerate(_SC_TABLES):
        @pl.when(jnp.logical_and(wid >= wstart, wid < wstart + nworkers))
        def _(t=t, wstart=wstart):
            _sc_worker_copy(srcs[t], dsts[t], wid, wstart, bufs, sems)


def _sc_copy(author, field):
    mesh = plsc.VectorSubcoreMesh(core_axis_name="c", subcore_axis_name="s")
    run = pl.kernel(
        _sc_body,
        out_type=(
            jax.ShapeDtypeStruct(author.shape, author.dtype),
            jax.ShapeDtypeStruct(field.shape, field.dtype),
        ),
        mesh=mesh,
        scratch_types=[
            pltpu.VMEM((_SC_CHUNK, _EMBED), jnp.float32),
            pltpu.VMEM((_SC_CHUNK, _EMBED), jnp.float32),
            pltpu.SemaphoreType.DMA,
            pltpu.SemaphoreType.DMA,
            pltpu.SemaphoreType.DMA,
            pltpu.SemaphoreType.DMA,
        ],
    )
    return run(author, field)


def kernel(embed_paper, embed_author, embed_field):
    paper_out = _tc_copy(embed_paper)
    author_out, field_out = _sc_copy(embed_author, embed_field)
    return (paper_out, author_out, field_out)


# speedup vs baseline: 1.1454x; 1.0007x over previous
import jax
import jax.numpy as jnp
from jax import lax
from jax.experimental import pallas as pl
from jax.experimental.pallas import tpu as pltpu
from jax.experimental.pallas import tpu_sc as plsc
from jax.experimental import compute_on

_EMBED = 128


_TC_CHUNK = 12500
_TC_SLOTS = 4


def _tc_pipeline(src, dst, *scratch):
    bufs = scratch[:_TC_SLOTS]
    sins = scratch[_TC_SLOTS:2 * _TC_SLOTS]
    souts = scratch[2 * _TC_SLOTS:]
    n = src.shape[0] // _TC_CHUNK
    depth = _TC_SLOTS // 2

    def in_copy(i):
        return pltpu.make_async_copy(
            src.at[pl.ds(i * _TC_CHUNK, _TC_CHUNK), :],
            bufs[i % _TC_SLOTS],
            sins[i % _TC_SLOTS],
        )

    def out_copy(i):
        return pltpu.make_async_copy(
            bufs[i % _TC_SLOTS],
            dst.at[pl.ds(i * _TC_CHUNK, _TC_CHUNK), :],
            souts[i % _TC_SLOTS],
        )

    for i in range(min(depth, n)):
        in_copy(i).start()
    for i in range(n):
        j = i + depth
        if j < n:
            if j - _TC_SLOTS >= 0:
                out_copy(j - _TC_SLOTS).wait()
            in_copy(j).start()
        in_copy(i).wait()
        out_copy(i).start()
    for i in range(max(0, n - 2 * depth), n):
        out_copy(i).wait()


def _tc_copy(x):
    return pl.pallas_call(
        _tc_pipeline,
        in_specs=[pl.BlockSpec(memory_space=pltpu.MemorySpace.HBM)],
        out_specs=pl.BlockSpec(memory_space=pltpu.MemorySpace.HBM),
        out_shape=jax.ShapeDtypeStruct(x.shape, x.dtype),
        scratch_shapes=(
            [pltpu.VMEM((_TC_CHUNK, _EMBED), jnp.float32)] * _TC_SLOTS
            + [pltpu.SemaphoreType.DMA] * (2 * _TC_SLOTS)
        ),
    )(x)


_SC_ROWS_W = 2000
_SC_CHUNK = 200
_SC_NCHUNK = _SC_ROWS_W // _SC_CHUNK
_SC_TABLES = ((0, 25), (25, 5))


def _sc_worker_copy(src, dst, wid, wstart, bufs, sems):
    base = (wid - wstart) * _SC_ROWS_W

    def in_copy(k):
        return pltpu.make_async_copy(
            src.at[pl.ds(base + k * _SC_CHUNK, _SC_CHUNK), :],
            bufs[k % 2],
            sems[k % 2],
        )

    def out_copy(k):
        return pltpu.make_async_copy(
            bufs[k % 2],
            dst.at[pl.ds(base + k * _SC_CHUNK, _SC_CHUNK), :],
            sems[2 + (k % 2)],
        )

    in_copy(0).start()
    for k in range(_SC_NCHUNK):
        if k + 1 < _SC_NCHUNK:
            if k >= 1:
                out_copy(k - 1).wait()
            in_copy(k + 1).start()
        in_copy(k).wait()
        out_copy(k).start()
    if _SC_NCHUNK >= 2:
        out_copy(_SC_NCHUNK - 2).wait()
    out_copy(_SC_NCHUNK - 1).wait()


def _sc_body(a_in, f_in, a_out, f_out, buf0, buf1, s0, s1, s2, s3):
    wid = lax.axis_index("s") * 2 + lax.axis_index("c")
    bufs = (buf0, buf1)
    sems = (s0, s1, s2, s3)
    srcs = (a_in, f_in)
    dsts = (a_out, f_out)
    for t, (wstart, nworkers) in enumerate(_SC_TABLES):
        @pl.when(jnp.logical_and(wid >= wstart, wid < wstart + nworkers))
        def _(t=t, wstart=wstart):
            _sc_worker_copy(srcs[t], dsts[t], wid, wstart, bufs, sems)


def _sc_copy(author, field):
    mesh = plsc.VectorSubcoreMesh(core_axis_name="c", subcore_axis_name="s")
    run = pl.kernel(
        _sc_body,
        out_type=(
            jax.ShapeDtypeStruct(author.shape, author.dtype),
            jax.ShapeDtypeStruct(field.shape, field.dtype),
        ),
        mesh=mesh,
        scratch_types=[
            pltpu.VMEM((_SC_CHUNK, _EMBED), jnp.float32),
            pltpu.VMEM((_SC_CHUNK, _EMBED), jnp.float32),
            pltpu.SemaphoreType.DMA,
            pltpu.SemaphoreType.DMA,
            pltpu.SemaphoreType.DMA,
            pltpu.SemaphoreType.DMA,
        ],
    )
    with compute_on.compute_on("tpu_sparsecore"):
        return run(author, field)


def kernel(embed_paper, embed_author, embed_field):
    author_out, field_out = _sc_copy(embed_author, embed_field)
    paper_out = _tc_copy(embed_paper)
    return (paper_out, author_out, field_out)


# speedup vs baseline: 1.5515x; 1.3545x over previous
import jax
import jax.numpy as jnp
from jax.experimental import pallas as pl
from jax.experimental.pallas import tpu as pltpu

_N_PAPER, _N_AUTHOR, _N_FIELD = 100000, 50000, 10000
_EMBED = 128
_CHUNK = 25000
_SLOTS = 4


def _chunk_list():
    chunks = []
    for t, n in enumerate((_N_PAPER, _N_AUTHOR, _N_FIELD)):
        off = 0
        while off < n:
            rows = min(_CHUNK, n - off)
            chunks.append((t, off, rows))
            off += rows
    return chunks


def _dma_pipeline(p_in, a_in, f_in, p_out, a_out, f_out, *scratch):
    bufs = scratch[:_SLOTS]
    sins = scratch[_SLOTS:2 * _SLOTS]
    souts = scratch[2 * _SLOTS:]
    srcs = (p_in, a_in, f_in)
    dsts = (p_out, a_out, f_out)
    chunks = _chunk_list()
    n = len(chunks)

    def in_copy(i):
        t, off, rows = chunks[i]
        return pltpu.make_async_copy(
            srcs[t].at[pl.ds(off, rows), :],
            bufs[i % _SLOTS].at[pl.ds(0, rows), :],
            sins[i % _SLOTS],
        )

    def out_copy(i):
        t, off, rows = chunks[i]
        return pltpu.make_async_copy(
            bufs[i % _SLOTS].at[pl.ds(0, rows), :],
            dsts[t].at[pl.ds(off, rows), :],
            souts[i % _SLOTS],
        )

    depth = _SLOTS // 2
    for i in range(min(depth, n)):
        in_copy(i).start()
    for i in range(n):
        j = i + depth
        if j < n:
            if j - _SLOTS >= 0:
                out_copy(j - _SLOTS).wait()
            in_copy(j).start()
        in_copy(i).wait()
        out_copy(i).start()
    for i in range(max(0, n - 2 * depth), n):
        out_copy(i).wait()


def kernel(embed_paper, embed_author, embed_field):
    return pl.pallas_call(
        _dma_pipeline,
        in_specs=[pl.BlockSpec(memory_space=pltpu.MemorySpace.HBM)] * 3,
        out_specs=(pl.BlockSpec(memory_space=pltpu.MemorySpace.HBM),) * 3,
        out_shape=tuple(
            jax.ShapeDtypeStruct(x.shape, x.dtype)
            for x in (embed_paper, embed_author, embed_field)
        ),
        scratch_shapes=(
            [pltpu.VMEM((_CHUNK, _EMBED), jnp.float32)] * _SLOTS
            + [pltpu.SemaphoreType.DMA] * (2 * _SLOTS)
        ),
    )(embed_paper, embed_author, embed_field)
